# baseline (speedup 1.0000x reference)
import functools

import jax
import jax.numpy as jnp
from jax import lax
from jax.experimental import pallas as pl
from jax.experimental.pallas import tpu as pltpu
from jax.experimental.pallas import tpu_sc as plsc

N = 10000
E = 320000
D = 128
DH = D // 2
STEPS = 10
OUT = 3
EPS = 1e-5

NC = 2
NS = 16
CHUNK = 128
NBUF = 4
GRP = 16
NGROUP = 10
NCHUNK = GRP * NGROUP
EPW = NCHUNK * CHUNK
EP = EPW * NS
ACC_ROWS = 10240
OUT_STRIPE = 624
HBM_CHUNK = 3


def _idx_group_cp(send_hbm, recv_hbm, sendbuf, recvbuf, sems, s, g):
    slot = lax.rem(g, 3)
    sem = sems.at[NBUF + slot]
    scp = pltpu.make_async_copy(
        send_hbm.at[pl.ds(s * EPW + g * GRP * CHUNK, GRP * CHUNK)],
        sendbuf.at[slot], sem)
    rcp = pltpu.make_async_copy(
        recv_hbm.at[s, pl.ds(g * GRP, GRP)], recvbuf.at[slot], sem)
    return scp, rcp


def _sc_agg_body(g0_hbm, g1_hbm, send_hbm, recv_hbm, out_hbm,
                 sendbuf, recvbuf, rows, hsp, accum, sems):
    c = lax.axis_index("c")
    s = lax.axis_index("s")

    for g0 in (0, 1):
        scp, rcp = _idx_group_cp(send_hbm, recv_hbm, sendbuf, recvbuf,
                                 sems, s, g0)
        scp.start()
        rcp.start()

    stage_sem = sems.at[NBUF + 3]
    sb = s * OUT_STRIPE
    tb = NS * OUT_STRIPE
    tn = N - NS * OUT_STRIPE

    @pl.when(c == 0)
    def _stage0():
        pltpu.async_copy(g0_hbm.at[pl.ds(sb, OUT_STRIPE)],
                         hsp.at[pl.ds(sb, OUT_STRIPE)], stage_sem)

        @pl.when(s == NS - 1)
        def _t0():
            pltpu.async_copy(g0_hbm.at[pl.ds(tb, tn)],
                             hsp.at[pl.ds(tb, tn)], stage_sem)

    @pl.when(c == 1)
    def _stage1():
        pltpu.async_copy(g1_hbm.at[pl.ds(sb, OUT_STRIPE)],
                         hsp.at[pl.ds(sb, OUT_STRIPE)], stage_sem)

        @pl.when(s == NS - 1)
        def _t1():
            pltpu.async_copy(g1_hbm.at[pl.ds(tb, tn)],
                             hsp.at[pl.ds(tb, tn)], stage_sem)

    zeros16 = jnp.zeros((16,), jnp.float32)

    def _zrow(i, carry):
        for j in range(DH // 16):
            rows[0, i, pl.ds(j * 16, 16)] = zeros16
        return carry

    lax.fori_loop(0, CHUNK, _zrow, 0)

    zbase = s * (ACC_ROWS // NS)
    zsem = sems.at[NBUF + 4]
    zcps = [pltpu.make_async_copy(
        rows.at[0], accum.at[pl.ds(zbase + k * CHUNK, CHUNK)], zsem)
        for k in range((ACC_ROWS // NS) // CHUNK)]
    for cp in zcps:
        cp.start()
    for cp in zcps:
        cp.wait()

    pltpu.make_async_copy(g0_hbm.at[pl.ds(sb, OUT_STRIPE)],
                          hsp.at[pl.ds(sb, OUT_STRIPE)], stage_sem).wait()

    @pl.when(s == NS - 1)
    def _wait_tail():
        pltpu.make_async_copy(g0_hbm.at[pl.ds(tb, tn)],
                              hsp.at[pl.ds(tb, tn)], stage_sem).wait()

    scp0, rcp0 = _idx_group_cp(send_hbm, recv_hbm, sendbuf, recvbuf,
                               sems, s, 0)
    scp0.wait()
    rcp0.wait()

    plsc.subcore_barrier()

    def _gather_cp(slot, off, b):
        return pltpu.make_async_copy(
            hsp.at[sendbuf.at[slot].at[pl.ds(off * CHUNK, CHUNK)]],
            rows.at[b], sems.at[b])

    def _start_gather(slot, off, b):
        idx = sendbuf.at[slot].at[pl.ds(off * CHUNK, CHUNK)]
        if off % NBUF == HBM_CHUNK:

            @pl.when(c == 0)
            def _h0():
                pltpu.make_async_copy(g0_hbm.at[idx], rows.at[b],
                                      sems.at[b]).start()

            @pl.when(c == 1)
            def _h1():
                pltpu.make_async_copy(g1_hbm.at[idx], rows.at[b],
                                      sems.at[b]).start()

        else:
            _gather_cp(slot, off, b).start()

    for b in range(NBUF):
        _start_gather(0, b, b)

    def _group(g, carry):
        gm = lax.rem(g, 3)
        gp1 = lax.rem(g + 1, 3)

        @pl.when(g < NGROUP - 1)
        def _wait_next_idx():
            scp, rcp = _idx_group_cp(send_hbm, recv_hbm, sendbuf, recvbuf,
                                     sems, s, g + 1)
            scp.wait()
            rcp.wait()

        @pl.when(g < NGROUP - 2)
        def _issue_idx():
            scp, rcp = _idx_group_cp(send_hbm, recv_hbm, sendbuf, recvbuf,
                                     sems, s, g + 2)
            scp.start()
            rcp.start()

        for r in range(GRP):
            b = r % NBUF
            _gather_cp(gm, r, b).wait()
            pltpu.sync_copy(rows.at[b], accum.at[recvbuf.at[gm, r]], add=True)
            rn = r + NBUF
            if rn < GRP:
                _start_gather(gm, rn, b)
            else:

                @pl.when(g < NGROUP - 1)
                def _issue_wrap():
                    _start_gather(gp1, rn - GRP, b)

        return carry

    lax.fori_loop(0, NGROUP, _group, 0)

    plsc.subcore_barrier()

    ob = s * OUT_STRIPE
    pltpu.sync_copy(accum.at[pl.ds(ob, OUT_STRIPE)],
                    out_hbm.at[c, pl.ds(ob, OUT_STRIPE)])

    @pl.when(s == NS - 1)
    def _tail():
        pltpu.sync_copy(accum.at[pl.ds(tb, tn)],
                        out_hbm.at[c, pl.ds(tb, tn)])


@functools.cache
def _sc_agg():
    return pl.kernel(
        _sc_agg_body,
        out_type=jax.ShapeDtypeStruct((NC, N, DH), jnp.float32),
        mesh=plsc.VectorSubcoreMesh(core_axis_name="c", subcore_axis_name="s"),
        scratch_types=[
            pltpu.VMEM((3, GRP * CHUNK), jnp.int32),
            pltpu.VMEM((3, GRP, CHUNK), jnp.int32),
            pltpu.VMEM((NBUF, CHUNK, DH), jnp.float32),
            pltpu.VMEM_SHARED((N, DH), jnp.float32),
            pltpu.VMEM_SHARED((ACC_ROWS, DH), jnp.float32),
            pltpu.SemaphoreType.DMA((NBUF + 5,)),
        ],
        compiler_params=pltpu.CompilerParams(use_tc_tiling_on_sc=False),
        name="sc_gather_segsum",
    )


def _ln(u, g, beta):
    mu = jnp.mean(u, axis=-1, keepdims=True)
    var = jnp.mean((u - mu) * (u - mu), axis=-1, keepdims=True)
    return (u - mu) * lax.rsqrt(var + EPS) * g + beta


def _enc_body(x_ref, w1_ref, b1_ref, w2_ref, b2_ref, g_ref, beta_ref,
              o_ref, og0_ref, og1_ref):
    t = jnp.maximum(
        jnp.dot(x_ref[...], w1_ref[...], preferred_element_type=jnp.float32)
        + b1_ref[...], 0.0)
    u = jnp.dot(t, w2_ref[...], preferred_element_type=jnp.float32) + b2_ref[...]
    h = _ln(u, g_ref[...], beta_ref[...])
    o_ref[...] = h
    og0_ref[...] = h[:, :DH]
    og1_ref[...] = h[:, DH:]


def _step_core(h_ref, agg_ref, w1h_ref, w1a_ref, b1_ref, w2_ref,
               b2_ref, g_ref, beta_ref):
    h = h_ref[...]
    agg = jnp.concatenate([agg_ref[0], agg_ref[1]], axis=-1)
    t = jnp.maximum(
        jnp.dot(h, w1h_ref[...], preferred_element_type=jnp.float32)
        + jnp.dot(agg, w1a_ref[...], preferred_element_type=jnp.float32)
        + b1_ref[...], 0.0)
    u = jnp.dot(t, w2_ref[...], preferred_element_type=jnp.float32) + b2_ref[...]
    return h + _ln(u, g_ref[...], beta_ref[...])


def _step_body(h_ref, agg_ref, w1h_ref, w1a_ref, b1_ref, w2_ref,
               b2_ref, g_ref, beta_ref, o_ref, og0_ref, og1_ref):
    hn = _step_core(h_ref, agg_ref, w1h_ref, w1a_ref, b1_ref, w2_ref,
                    b2_ref, g_ref, beta_ref)
    o_ref[...] = hn
    og0_ref[...] = hn[:, :DH]
    og1_ref[...] = hn[:, DH:]


def _final_body(h_ref, agg_ref, w1h_ref, w1a_ref, b1_ref, w2_ref,
                b2_ref, g_ref, beta_ref, dw1_ref, db1_ref, dw2_ref,
                db2_ref, o_ref):
    hn = _step_core(h_ref, agg_ref, w1h_ref, w1a_ref, b1_ref, w2_ref,
                    b2_ref, g_ref, beta_ref)
    t = jnp.maximum(
        jnp.dot(hn, dw1_ref[...], preferred_element_type=jnp.float32)
        + db1_ref[...], 0.0)
    o_ref[...] = (
        jnp.dot(t, dw2_ref[...], preferred_element_type=jnp.float32)
        + db2_ref[...])


_ROW_BLK = 1000
_GRID = N // _ROW_BLK


def _row_spec():
    return pl.BlockSpec((_ROW_BLK, D), lambda i: (i, 0))


def _g_spec():
    return pl.BlockSpec((_ROW_BLK, DH), lambda i: (i, 0))


def _agg_spec():
    return pl.BlockSpec((2, _ROW_BLK, DH), lambda i: (0, i, 0))


def _full_spec(r):
    return pl.BlockSpec((r, D), lambda i: (0, 0))


_h_shape = jax.ShapeDtypeStruct((N, D), jnp.float32)
_g_shape = jax.ShapeDtypeStruct((N, DH), jnp.float32)

_enc_call = pl.pallas_call(
    _enc_body,
    grid=(_GRID,),
    in_specs=[_row_spec(), _full_spec(D), _full_spec(1), _full_spec(D),
              _full_spec(1), _full_spec(1), _full_spec(1)],
    out_specs=[_row_spec(), _g_spec(), _g_spec()],
    out_shape=[_h_shape, _g_shape, _g_shape],
)

_step_call = pl.pallas_call(
    _step_body,
    grid=(_GRID,),
    in_specs=[_row_spec(), _agg_spec(),
              _full_spec(D), _full_spec(D), _full_spec(1), _full_spec(D),
              _full_spec(1), _full_spec(1), _full_spec(1)],
    out_specs=[_row_spec(), _g_spec(), _g_spec()],
    out_shape=[_h_shape, _g_shape, _g_shape],
)

_final_call = pl.pallas_call(
    _final_body,
    grid=(_GRID,),
    in_specs=[_row_spec(), _agg_spec(),
              _full_spec(D), _full_spec(D), _full_spec(1), _full_spec(D),
              _full_spec(1), _full_spec(1), _full_spec(1),
              _full_spec(D), _full_spec(1), _full_spec(D), _full_spec(1)],
    out_specs=_row_spec(),
    out_shape=_h_shape,
)


def kernel(x, edge_index, enc_W1, enc_b1, enc_W2, enc_b2, enc_g, enc_beta,
           Pw1, Pb1, Pw2, Pb2, Pg, Pbeta, Dw1, Db1, Dw2, Db2):
    senders = edge_index[0]
    receivers = edge_index[1]
    pad = EP - E
    send_p = jnp.concatenate([senders, jnp.zeros((pad,), jnp.int32)])
    recv_p = jnp.concatenate([receivers, jnp.full((pad,), N, jnp.int32)])
    recv3d = recv_p.reshape(NS, NCHUNK, CHUNK)

    r2 = lambda v: v.reshape(1, D)

    h, g0, g1 = _enc_call(x, enc_W1, r2(enc_b1), enc_W2, r2(enc_b2),
                          r2(enc_g), r2(enc_beta))

    dw2_pad = jnp.zeros((D, D), jnp.float32).at[:, :OUT].set(Dw2)
    db2_pad = jnp.zeros((1, D), jnp.float32).at[0, :OUT].set(Db2)

    for i in range(STEPS):
        agg = _sc_agg()(g0, g1, send_p, recv3d)
        wargs = (Pw1[i, :D], Pw1[i, D:], Pb1[i].reshape(1, D), Pw2[i],
                 Pb2[i].reshape(1, D), Pg[i].reshape(1, D),
                 Pbeta[i].reshape(1, D))
        if i < STEPS - 1:
            h, g0, g1 = _step_call(h, agg, *wargs)
        else:
            out_pad = _final_call(h, agg, *wargs, Dw1, r2(Db1), dw2_pad,
                                  db2_pad)
    return out_pad[:, :OUT]


# speedup vs baseline: 1.1568x; 1.1568x over previous
import functools

import jax
import jax.numpy as jnp
from jax import lax
from jax.experimental import pallas as pl
from jax.experimental.pallas import tpu as pltpu
from jax.experimental.pallas import tpu_sc as plsc

N = 10000
E = 320000
D = 128
DH = D // 2
STEPS = 10
OUT = 3
EPS = 1e-5

NC = 2
NS = 16
CHUNK = 128
NBUF = 4
GRP = 16
NGROUP = 10
NCHUNK = GRP * NGROUP
EPW = NCHUNK * CHUNK
EP = EPW * NS
ACC_ROWS = 10240
OUT_STRIPE = 624
HBM_CHUNK = 3


def _idx_group_cp(send_hbm, recv_hbm, sendbuf, recvbuf, sems, s, g):
    slot = lax.rem(g, 3)
    sem = sems.at[NBUF + slot]
    scp = pltpu.make_async_copy(
        send_hbm.at[pl.ds(s * EPW + g * GRP * CHUNK, GRP * CHUNK)],
        sendbuf.at[slot], sem)
    rcp = pltpu.make_async_copy(
        recv_hbm.at[s, pl.ds(g * GRP, GRP)], recvbuf.at[slot], sem)
    return scp, rcp


def _sc_agg_body(g0_hbm, g1_hbm, send_hbm, recv_hbm, out_hbm,
                 sendbuf, recvbuf, rows, hsp, accum, sems):
    c = lax.axis_index("c")
    s = lax.axis_index("s")

    for g0 in (0, 1):
        scp, rcp = _idx_group_cp(send_hbm, recv_hbm, sendbuf, recvbuf,
                                 sems, s, g0)
        scp.start()
        rcp.start()

    stage_sem = sems.at[NBUF + 3]
    sb = s * OUT_STRIPE
    tb = NS * OUT_STRIPE
    tn = N - NS * OUT_STRIPE

    @pl.when(c == 0)
    def _stage0():
        pltpu.async_copy(g0_hbm.at[pl.ds(sb, OUT_STRIPE)],
                         hsp.at[pl.ds(sb, OUT_STRIPE)], stage_sem)

        @pl.when(s == NS - 1)
        def _t0():
            pltpu.async_copy(g0_hbm.at[pl.ds(tb, tn)],
                             hsp.at[pl.ds(tb, tn)], stage_sem)

    @pl.when(c == 1)
    def _stage1():
        pltpu.async_copy(g1_hbm.at[pl.ds(sb, OUT_STRIPE)],
                         hsp.at[pl.ds(sb, OUT_STRIPE)], stage_sem)

        @pl.when(s == NS - 1)
        def _t1():
            pltpu.async_copy(g1_hbm.at[pl.ds(tb, tn)],
                             hsp.at[pl.ds(tb, tn)], stage_sem)

    zeros16 = jnp.zeros((16,), jnp.float32)

    def _zrow(i, carry):
        for j in range(DH // 16):
            rows[0, i, pl.ds(j * 16, 16)] = zeros16
        return carry

    lax.fori_loop(0, CHUNK, _zrow, 0)

    zbase = s * (ACC_ROWS // NS)
    zsem = sems.at[NBUF + 4]
    zcps = [pltpu.make_async_copy(
        rows.at[0], accum.at[pl.ds(zbase + k * CHUNK, CHUNK)], zsem)
        for k in range((ACC_ROWS // NS) // CHUNK)]
    for cp in zcps:
        cp.start()
    for cp in zcps:
        cp.wait()

    pltpu.make_async_copy(g0_hbm.at[pl.ds(sb, OUT_STRIPE)],
                          hsp.at[pl.ds(sb, OUT_STRIPE)], stage_sem).wait()

    @pl.when(s == NS - 1)
    def _wait_tail():
        pltpu.make_async_copy(g0_hbm.at[pl.ds(tb, tn)],
                              hsp.at[pl.ds(tb, tn)], stage_sem).wait()

    scp0, rcp0 = _idx_group_cp(send_hbm, recv_hbm, sendbuf, recvbuf,
                               sems, s, 0)
    scp0.wait()
    rcp0.wait()

    plsc.subcore_barrier()

    def _gather_cp(slot, off, b):
        return pltpu.make_async_copy(
            hsp.at[sendbuf.at[slot].at[pl.ds(off * CHUNK, CHUNK)]],
            rows.at[b], sems.at[b])

    def _start_gather(slot, off, b):
        _gather_cp(slot, off, b).start()

    for b in range(NBUF):
        _start_gather(0, b, b)

    def _group(g, carry):
        gm = lax.rem(g, 3)
        gp1 = lax.rem(g + 1, 3)

        @pl.when(g < NGROUP - 1)
        def _wait_next_idx():
            scp, rcp = _idx_group_cp(send_hbm, recv_hbm, sendbuf, recvbuf,
                                     sems, s, g + 1)
            scp.wait()
            rcp.wait()

        @pl.when(g < NGROUP - 2)
        def _issue_idx():
            scp, rcp = _idx_group_cp(send_hbm, recv_hbm, sendbuf, recvbuf,
                                     sems, s, g + 2)
            scp.start()
            rcp.start()

        for r in range(GRP):
            b = r % NBUF
            _gather_cp(gm, r, b).wait()
            pltpu.sync_copy(rows.at[b], accum.at[recvbuf.at[gm, r]], add=True)
            rn = r + NBUF
            if rn < GRP:
                _start_gather(gm, rn, b)
            else:

                @pl.when(g < NGROUP - 1)
                def _issue_wrap():
                    _start_gather(gp1, rn - GRP, b)

        return carry

    lax.fori_loop(0, NGROUP, _group, 0)

    plsc.subcore_barrier()

    ob = s * OUT_STRIPE
    pltpu.sync_copy(accum.at[pl.ds(ob, OUT_STRIPE)],
                    out_hbm.at[c, pl.ds(ob, OUT_STRIPE)])

    @pl.when(s == NS - 1)
    def _tail():
        pltpu.sync_copy(accum.at[pl.ds(tb, tn)],
                        out_hbm.at[c, pl.ds(tb, tn)])


@functools.cache
def _sc_agg():
    return pl.kernel(
        _sc_agg_body,
        out_type=jax.ShapeDtypeStruct((NC, N, DH), jnp.float32),
        mesh=plsc.VectorSubcoreMesh(core_axis_name="c", subcore_axis_name="s"),
        scratch_types=[
            pltpu.VMEM((3, GRP * CHUNK), jnp.int32),
            pltpu.VMEM((3, GRP, CHUNK), jnp.int32),
            pltpu.VMEM((NBUF, CHUNK, DH), jnp.float32),
            pltpu.VMEM_SHARED((N, DH), jnp.float32),
            pltpu.VMEM_SHARED((ACC_ROWS, DH), jnp.float32),
            pltpu.SemaphoreType.DMA((NBUF + 5,)),
        ],
        compiler_params=pltpu.CompilerParams(use_tc_tiling_on_sc=False),
        name="sc_gather_segsum",
    )


def _ln(u, g, beta):
    mu = jnp.mean(u, axis=-1, keepdims=True)
    var = jnp.mean((u - mu) * (u - mu), axis=-1, keepdims=True)
    return (u - mu) * lax.rsqrt(var + EPS) * g + beta


def _enc_body(x_ref, w1_ref, b1_ref, w2_ref, b2_ref, g_ref, beta_ref,
              o_ref, og0_ref, og1_ref):
    t = jnp.maximum(
        jnp.dot(x_ref[...], w1_ref[...], preferred_element_type=jnp.float32)
        + b1_ref[...], 0.0)
    u = jnp.dot(t, w2_ref[...], preferred_element_type=jnp.float32) + b2_ref[...]
    h = _ln(u, g_ref[...], beta_ref[...])
    o_ref[...] = h
    og0_ref[...] = h[:, :DH]
    og1_ref[...] = h[:, DH:]


def _step_core(h_ref, agg_ref, w1h_ref, w1a_ref, b1_ref, w2_ref,
               b2_ref, g_ref, beta_ref):
    h = h_ref[...]
    agg = jnp.concatenate([agg_ref[0], agg_ref[1]], axis=-1)
    t = jnp.maximum(
        jnp.dot(h, w1h_ref[...], preferred_element_type=jnp.float32)
        + jnp.dot(agg, w1a_ref[...], preferred_element_type=jnp.float32)
        + b1_ref[...], 0.0)
    u = jnp.dot(t, w2_ref[...], preferred_element_type=jnp.float32) + b2_ref[...]
    return h + _ln(u, g_ref[...], beta_ref[...])


def _step_body(h_ref, agg_ref, w1h_ref, w1a_ref, b1_ref, w2_ref,
               b2_ref, g_ref, beta_ref, o_ref, og0_ref, og1_ref):
    hn = _step_core(h_ref, agg_ref, w1h_ref, w1a_ref, b1_ref, w2_ref,
                    b2_ref, g_ref, beta_ref)
    o_ref[...] = hn
    og0_ref[...] = hn[:, :DH]
    og1_ref[...] = hn[:, DH:]


def _final_body(h_ref, agg_ref, w1h_ref, w1a_ref, b1_ref, w2_ref,
                b2_ref, g_ref, beta_ref, dw1_ref, db1_ref, dw2_ref,
                db2_ref, o_ref):
    hn = _step_core(h_ref, agg_ref, w1h_ref, w1a_ref, b1_ref, w2_ref,
                    b2_ref, g_ref, beta_ref)
    t = jnp.maximum(
        jnp.dot(hn, dw1_ref[...], preferred_element_type=jnp.float32)
        + db1_ref[...], 0.0)
    o_ref[...] = (
        jnp.dot(t, dw2_ref[...], preferred_element_type=jnp.float32)
        + db2_ref[...])


_ROW_BLK = 1000
_GRID = N // _ROW_BLK


def _row_spec():
    return pl.BlockSpec((_ROW_BLK, D), lambda i: (i, 0))


def _g_spec():
    return pl.BlockSpec((_ROW_BLK, DH), lambda i: (i, 0))


def _agg_spec():
    return pl.BlockSpec((2, _ROW_BLK, DH), lambda i: (0, i, 0))


def _full_spec(r):
    return pl.BlockSpec((r, D), lambda i: (0, 0))


_h_shape = jax.ShapeDtypeStruct((N, D), jnp.float32)
_g_shape = jax.ShapeDtypeStruct((N, DH), jnp.float32)

_enc_call = pl.pallas_call(
    _enc_body,
    grid=(_GRID,),
    in_specs=[_row_spec(), _full_spec(D), _full_spec(1), _full_spec(D),
              _full_spec(1), _full_spec(1), _full_spec(1)],
    out_specs=[_row_spec(), _g_spec(), _g_spec()],
    out_shape=[_h_shape, _g_shape, _g_shape],
)

_step_call = pl.pallas_call(
    _step_body,
    grid=(_GRID,),
    in_specs=[_row_spec(), _agg_spec(),
              _full_spec(D), _full_spec(D), _full_spec(1), _full_spec(D),
              _full_spec(1), _full_spec(1), _full_spec(1)],
    out_specs=[_row_spec(), _g_spec(), _g_spec()],
    out_shape=[_h_shape, _g_shape, _g_shape],
)

_final_call = pl.pallas_call(
    _final_body,
    grid=(_GRID,),
    in_specs=[_row_spec(), _agg_spec(),
              _full_spec(D), _full_spec(D), _full_spec(1), _full_spec(D),
              _full_spec(1), _full_spec(1), _full_spec(1),
              _full_spec(D), _full_spec(1), _full_spec(D), _full_spec(1)],
    out_specs=_row_spec(),
    out_shape=_h_shape,
)


def kernel(x, edge_index, enc_W1, enc_b1, enc_W2, enc_b2, enc_g, enc_beta,
           Pw1, Pb1, Pw2, Pb2, Pg, Pbeta, Dw1, Db1, Dw2, Db2):
    senders = edge_index[0]
    receivers = edge_index[1]
    pad = EP - E
    send_p = jnp.concatenate([senders, jnp.zeros((pad,), jnp.int32)])
    recv_p = jnp.concatenate([receivers, jnp.full((pad,), N, jnp.int32)])
    recv3d = recv_p.reshape(NS, NCHUNK, CHUNK)

    r2 = lambda v: v.reshape(1, D)

    h, g0, g1 = _enc_call(x, enc_W1, r2(enc_b1), enc_W2, r2(enc_b2),
                          r2(enc_g), r2(enc_beta))

    dw2_pad = jnp.zeros((D, D), jnp.float32).at[:, :OUT].set(Dw2)
    db2_pad = jnp.zeros((1, D), jnp.float32).at[0, :OUT].set(Db2)

    for i in range(STEPS):
        agg = _sc_agg()(g0, g1, send_p, recv3d)
        wargs = (Pw1[i, :D], Pw1[i, D:], Pb1[i].reshape(1, D), Pw2[i],
                 Pb2[i].reshape(1, D), Pg[i].reshape(1, D),
                 Pbeta[i].reshape(1, D))
        if i < STEPS - 1:
            h, g0, g1 = _step_call(h, agg, *wargs)
        else:
            out_pad = _final_call(h, agg, *wargs, Dw1, r2(Db1), dw2_pad,
                                  db2_pad)
    return out_pad[:, :OUT]


# speedup vs baseline: 1.3228x; 1.1436x over previous
import functools

import jax
import jax.numpy as jnp
from jax import lax
from jax.experimental import pallas as pl
from jax.experimental.pallas import tpu as pltpu
from jax.experimental.pallas import tpu_sc as plsc

N = 10000
E = 320000
D = 128
DH = D // 2
STEPS = 10
OUT = 3
EPS = 1e-5

NC = 2
NS = 16
CHUNK = 128
NBUF = 4
GRP = 16
NGROUP = 10
NCHUNK = GRP * NGROUP
EPW = NCHUNK * CHUNK
EP = EPW * NS
ACC_ROWS = 10240
OUT_STRIPE = 624
HBM_CHUNK = 3


def _idx_group_cp(send_hbm, recv_hbm, sendbuf, recvbuf, sems, s, g):
    slot = lax.rem(g, 3)
    sem = sems.at[2 * NBUF + slot]
    scp = pltpu.make_async_copy(
        send_hbm.at[pl.ds(s * EPW + g * GRP * CHUNK, GRP * CHUNK)],
        sendbuf.at[slot], sem)
    rcp = pltpu.make_async_copy(
        recv_hbm.at[s, pl.ds(g * GRP, GRP)], recvbuf.at[slot], sem)
    return scp, rcp


def _sc_agg_body(g0_hbm, g1_hbm, send_hbm, recv_hbm, out_hbm,
                 sendbuf, recvbuf, rows, hsp, accum, sems):
    c = lax.axis_index("c")
    s = lax.axis_index("s")

    for g0 in (0, 1):
        scp, rcp = _idx_group_cp(send_hbm, recv_hbm, sendbuf, recvbuf,
                                 sems, s, g0)
        scp.start()
        rcp.start()

    stage_sem = sems.at[2 * NBUF + 3]
    sb = s * OUT_STRIPE
    tb = NS * OUT_STRIPE
    tn = N - NS * OUT_STRIPE

    @pl.when(c == 0)
    def _stage0():
        pltpu.async_copy(g0_hbm.at[pl.ds(sb, OUT_STRIPE)],
                         hsp.at[pl.ds(sb, OUT_STRIPE)], stage_sem)

        @pl.when(s == NS - 1)
        def _t0():
            pltpu.async_copy(g0_hbm.at[pl.ds(tb, tn)],
                             hsp.at[pl.ds(tb, tn)], stage_sem)

    @pl.when(c == 1)
    def _stage1():
        pltpu.async_copy(g1_hbm.at[pl.ds(sb, OUT_STRIPE)],
                         hsp.at[pl.ds(sb, OUT_STRIPE)], stage_sem)

        @pl.when(s == NS - 1)
        def _t1():
            pltpu.async_copy(g1_hbm.at[pl.ds(tb, tn)],
                             hsp.at[pl.ds(tb, tn)], stage_sem)

    zeros16 = jnp.zeros((16,), jnp.float32)

    def _zrow(i, carry):
        for j in range(DH // 16):
            rows[0, i, pl.ds(j * 16, 16)] = zeros16
        return carry

    lax.fori_loop(0, CHUNK, _zrow, 0)

    zbase = s * (ACC_ROWS // NS)
    zsem = sems.at[2 * NBUF + 4]
    zcps = [pltpu.make_async_copy(
        rows.at[0], accum.at[pl.ds(zbase + k * CHUNK, CHUNK)], zsem)
        for k in range((ACC_ROWS // NS) // CHUNK)]
    for cp in zcps:
        cp.start()
    for cp in zcps:
        cp.wait()

    pltpu.make_async_copy(g0_hbm.at[pl.ds(sb, OUT_STRIPE)],
                          hsp.at[pl.ds(sb, OUT_STRIPE)], stage_sem).wait()

    @pl.when(s == NS - 1)
    def _wait_tail():
        pltpu.make_async_copy(g0_hbm.at[pl.ds(tb, tn)],
                              hsp.at[pl.ds(tb, tn)], stage_sem).wait()

    scp0, rcp0 = _idx_group_cp(send_hbm, recv_hbm, sendbuf, recvbuf,
                               sems, s, 0)
    scp0.wait()
    rcp0.wait()

    plsc.subcore_barrier()

    GDEP = 2

    def _gather_cp(slot, off, b):
        return pltpu.make_async_copy(
            hsp.at[sendbuf.at[slot].at[pl.ds(off * CHUNK, CHUNK)]],
            rows.at[b], sems.at[b])

    def _scat_cp(slot, off, b):
        return pltpu.make_async_copy(
            rows.at[b], accum.at[recvbuf.at[slot, off]], sems.at[NBUF + b])

    def _start_scat(slot, off, b):
        pltpu.async_copy(rows.at[b], accum.at[recvbuf.at[slot, off]],
                         sems.at[NBUF + b], add=True)

    for b in range(GDEP):
        _gather_cp(0, b, b).start()

    def _group(g, carry):
        gm = lax.rem(g, 3)
        gp1 = lax.rem(g + 1, 3)

        @pl.when(g < NGROUP - 1)
        def _wait_next_idx():
            scp, rcp = _idx_group_cp(send_hbm, recv_hbm, sendbuf, recvbuf,
                                     sems, s, g + 1)
            scp.wait()
            rcp.wait()

        @pl.when(g < NGROUP - 2)
        def _issue_idx():
            scp, rcp = _idx_group_cp(send_hbm, recv_hbm, sendbuf, recvbuf,
                                     sems, s, g + 2)
            scp.start()
            rcp.start()

        for r in range(GRP):
            b = r % NBUF
            _gather_cp(gm, r, b).wait()
            _start_scat(gm, r, b)
            bn = (r + GDEP) % NBUF
            rp = r + GDEP - NBUF
            if rp >= 0:
                _scat_cp(gm, rp, bn).wait()
            else:

                @pl.when(g > 0)
                def _wait_prev_grp():
                    _scat_cp(lax.rem(g + 2, 3), rp + GRP, bn).wait()

            rn = r + GDEP
            if rn < GRP:
                _gather_cp(gm, rn, bn).start()
            else:

                @pl.when(g < NGROUP - 1)
                def _issue_wrap():
                    _gather_cp(gp1, rn - GRP, bn).start()

        return carry

    lax.fori_loop(0, NGROUP, _group, 0)

    for r in range(GRP - GDEP, GRP):
        _scat_cp((NGROUP - 1) % 3, r, r % NBUF).wait()

    plsc.subcore_barrier()

    ob = s * OUT_STRIPE
    pltpu.sync_copy(accum.at[pl.ds(ob, OUT_STRIPE)],
                    out_hbm.at[c, pl.ds(ob, OUT_STRIPE)])

    @pl.when(s == NS - 1)
    def _tail():
        pltpu.sync_copy(accum.at[pl.ds(tb, tn)],
                        out_hbm.at[c, pl.ds(tb, tn)])


@functools.cache
def _sc_agg():
    return pl.kernel(
        _sc_agg_body,
        out_type=jax.ShapeDtypeStruct((NC, N, DH), jnp.float32),
        mesh=plsc.VectorSubcoreMesh(core_axis_name="c", subcore_axis_name="s"),
        scratch_types=[
            pltpu.VMEM((3, GRP * CHUNK), jnp.int32),
            pltpu.VMEM((3, GRP, CHUNK), jnp.int32),
            pltpu.VMEM((NBUF, CHUNK, DH), jnp.float32),
            pltpu.VMEM_SHARED((N, DH), jnp.float32),
            pltpu.VMEM_SHARED((ACC_ROWS, DH), jnp.float32),
            pltpu.SemaphoreType.DMA((2 * NBUF + 5,)),
        ],
        compiler_params=pltpu.CompilerParams(use_tc_tiling_on_sc=False),
        name="sc_gather_segsum",
    )


def _ln(u, g, beta):
    mu = jnp.mean(u, axis=-1, keepdims=True)
    var = jnp.mean((u - mu) * (u - mu), axis=-1, keepdims=True)
    return (u - mu) * lax.rsqrt(var + EPS) * g + beta


def _enc_body(x_ref, w1_ref, b1_ref, w2_ref, b2_ref, g_ref, beta_ref,
              o_ref, og0_ref, og1_ref):
    t = jnp.maximum(
        jnp.dot(x_ref[...], w1_ref[...], preferred_element_type=jnp.float32)
        + b1_ref[...], 0.0)
    u = jnp.dot(t, w2_ref[...], preferred_element_type=jnp.float32) + b2_ref[...]
    h = _ln(u, g_ref[...], beta_ref[...])
    o_ref[...] = h
    og0_ref[...] = h[:, :DH]
    og1_ref[...] = h[:, DH:]


def _step_core(h_ref, agg_ref, w1h_ref, w1a_ref, b1_ref, w2_ref,
               b2_ref, g_ref, beta_ref):
    h = h_ref[...]
    agg = jnp.concatenate([agg_ref[0], agg_ref[1]], axis=-1)
    t = jnp.maximum(
        jnp.dot(h, w1h_ref[...], preferred_element_type=jnp.float32)
        + jnp.dot(agg, w1a_ref[...], preferred_element_type=jnp.float32)
        + b1_ref[...], 0.0)
    u = jnp.dot(t, w2_ref[...], preferred_element_type=jnp.float32) + b2_ref[...]
    return h + _ln(u, g_ref[...], beta_ref[...])


def _step_body(h_ref, agg_ref, w1h_ref, w1a_ref, b1_ref, w2_ref,
               b2_ref, g_ref, beta_ref, o_ref, og0_ref, og1_ref):
    hn = _step_core(h_ref, agg_ref, w1h_ref, w1a_ref, b1_ref, w2_ref,
                    b2_ref, g_ref, beta_ref)
    o_ref[...] = hn
    og0_ref[...] = hn[:, :DH]
    og1_ref[...] = hn[:, DH:]


def _final_body(h_ref, agg_ref, w1h_ref, w1a_ref, b1_ref, w2_ref,
                b2_ref, g_ref, beta_ref, dw1_ref, db1_ref, dw2_ref,
                db2_ref, o_ref):
    hn = _step_core(h_ref, agg_ref, w1h_ref, w1a_ref, b1_ref, w2_ref,
                    b2_ref, g_ref, beta_ref)
    t = jnp.maximum(
        jnp.dot(hn, dw1_ref[...], preferred_element_type=jnp.float32)
        + db1_ref[...], 0.0)
    o_ref[...] = (
        jnp.dot(t, dw2_ref[...], preferred_element_type=jnp.float32)
        + db2_ref[...])


_ROW_BLK = 1000
_GRID = N // _ROW_BLK


def _row_spec():
    return pl.BlockSpec((_ROW_BLK, D), lambda i: (i, 0))


def _g_spec():
    return pl.BlockSpec((_ROW_BLK, DH), lambda i: (i, 0))


def _agg_spec():
    return pl.BlockSpec((2, _ROW_BLK, DH), lambda i: (0, i, 0))


def _full_spec(r):
    return pl.BlockSpec((r, D), lambda i: (0, 0))


_h_shape = jax.ShapeDtypeStruct((N, D), jnp.float32)
_g_shape = jax.ShapeDtypeStruct((N, DH), jnp.float32)

_enc_call = pl.pallas_call(
    _enc_body,
    grid=(_GRID,),
    in_specs=[_row_spec(), _full_spec(D), _full_spec(1), _full_spec(D),
              _full_spec(1), _full_spec(1), _full_spec(1)],
    out_specs=[_row_spec(), _g_spec(), _g_spec()],
    out_shape=[_h_shape, _g_shape, _g_shape],
)

_step_call = pl.pallas_call(
    _step_body,
    grid=(_GRID,),
    in_specs=[_row_spec(), _agg_spec(),
              _full_spec(D), _full_spec(D), _full_spec(1), _full_spec(D),
              _full_spec(1), _full_spec(1), _full_spec(1)],
    out_specs=[_row_spec(), _g_spec(), _g_spec()],
    out_shape=[_h_shape, _g_shape, _g_shape],
)

_final_call = pl.pallas_call(
    _final_body,
    grid=(_GRID,),
    in_specs=[_row_spec(), _agg_spec(),
              _full_spec(D), _full_spec(D), _full_spec(1), _full_spec(D),
              _full_spec(1), _full_spec(1), _full_spec(1),
              _full_spec(D), _full_spec(1), _full_spec(D), _full_spec(1)],
    out_specs=_row_spec(),
    out_shape=_h_shape,
)


def kernel(x, edge_index, enc_W1, enc_b1, enc_W2, enc_b2, enc_g, enc_beta,
           Pw1, Pb1, Pw2, Pb2, Pg, Pbeta, Dw1, Db1, Dw2, Db2):
    senders = edge_index[0]
    receivers = edge_index[1]
    pad = EP - E
    send_p = jnp.concatenate([senders, jnp.zeros((pad,), jnp.int32)])
    recv_p = jnp.concatenate([receivers, jnp.full((pad,), N, jnp.int32)])
    recv3d = recv_p.reshape(NS, NCHUNK, CHUNK)

    r2 = lambda v: v.reshape(1, D)

    h, g0, g1 = _enc_call(x, enc_W1, r2(enc_b1), enc_W2, r2(enc_b2),
                          r2(enc_g), r2(enc_beta))

    dw2_pad = jnp.zeros((D, D), jnp.float32).at[:, :OUT].set(Dw2)
    db2_pad = jnp.zeros((1, D), jnp.float32).at[0, :OUT].set(Db2)

    for i in range(STEPS):
        agg = _sc_agg()(g0, g1, send_p, recv3d)
        wargs = (Pw1[i, :D], Pw1[i, D:], Pb1[i].reshape(1, D), Pw2[i],
                 Pb2[i].reshape(1, D), Pg[i].reshape(1, D),
                 Pbeta[i].reshape(1, D))
        if i < STEPS - 1:
            h, g0, g1 = _step_call(h, agg, *wargs)
        else:
            out_pad = _final_call(h, agg, *wargs, Dw1, r2(Db1), dw2_pad,
                                  db2_pad)
    return out_pad[:, :OUT]
